# Initial kernel scaffold; baseline (speedup 1.0000x reference)
import functools

import jax
import jax.numpy as jnp
from jax import lax
from jax.experimental import pallas as pl
from jax.experimental.pallas import tpu as pltpu
from jax.experimental.pallas import tpu_sc as plsc

_N = 10000
_D = 128
_E = 320000
_NSUB = 16
_NCORE = 2
_CHUNK = 128
_CPS = 79
_ROWS = _CPS * _NSUB * _NCORE
_EPAD = _ROWS * _CHUNK
_RPS = _N // _NSUB

_sc_mesh = plsc.VectorSubcoreMesh(core_axis_name="c", subcore_axis_name="s")


@functools.partial(
    pl.kernel,
    out_type=jax.ShapeDtypeStruct((_NCORE, _N, _D), jnp.float32),
    mesh=_sc_mesh,
    scratch_types=[
        pltpu.VMEM((_CPS, _CHUNK), jnp.int32),
        pltpu.VMEM((_CPS, _CHUNK), jnp.int32),
        pltpu.VMEM((_CHUNK, _D), jnp.float32),
        pltpu.VMEM_SHARED((_N + 1, _D), jnp.float32),
    ],
)
def _agg(x_hbm, src_hbm, dst_hbm, p_hbm, src_idx, dst_idx, rows, acc):
    cid = lax.axis_index("c")
    sid = lax.axis_index("s")
    cbase = (cid * _NSUB + sid) * _CPS
    pltpu.sync_copy(src_hbm.at[pl.ds(cbase, _CPS)], src_idx)
    pltpu.sync_copy(dst_hbm.at[pl.ds(cbase, _CPS)], dst_idx)
    rbase = sid * _RPS
    pltpu.sync_copy(x_hbm.at[pl.ds(rbase, _RPS)], acc.at[pl.ds(rbase, _RPS)])
    plsc.subcore_barrier()

    @pl.loop(0, _CPS)
    def _(j):
        pltpu.sync_copy(x_hbm.at[src_idx.at[j]], rows)
        pltpu.sync_copy(rows, acc.at[dst_idx.at[j]], add=True)

    plsc.subcore_barrier()
    pltpu.sync_copy(acc.at[pl.ds(rbase, _RPS)],
                    p_hbm.at[cid, pl.ds(rbase, _RPS)])


def _mlp_block(p_ref, x_ref, w1_ref, b1_ref, w2_ref, b2_ref, o_ref, *,
               final_relu):
    pb = p_ref[...]
    z = pb[0] + pb[1] - x_ref[...]
    t = jnp.dot(z, w1_ref[...], preferred_element_type=jnp.float32)
    t = jnp.maximum(t + b1_ref[...], 0.0)
    o = jnp.dot(t, w2_ref[...], preferred_element_type=jnp.float32)
    o = o + b2_ref[...]
    if final_relu:
        o = jnp.maximum(o, 0.0)
    o_ref[...] = o


_BLK = 1000


def _mlp(p, x, w1, b1, w2, b2, final_relu):
    return pl.pallas_call(
        functools.partial(_mlp_block, final_relu=final_relu),
        grid=(_N // _BLK,),
        in_specs=[
            pl.BlockSpec((_NCORE, _BLK, _D), lambda i: (0, i, 0)),
            pl.BlockSpec((_BLK, _D), lambda i: (i, 0)),
            pl.BlockSpec((_D, _D), lambda i: (0, 0)),
            pl.BlockSpec((1, _D), lambda i: (0, 0)),
            pl.BlockSpec((_D, _D), lambda i: (0, 0)),
            pl.BlockSpec((1, _D), lambda i: (0, 0)),
        ],
        out_specs=pl.BlockSpec((_BLK, _D), lambda i: (i, 0)),
        out_shape=jax.ShapeDtypeStruct((_N, _D), jnp.float32),
    )(p, x, w1, b1, w2, b2)


def kernel(x, edge_index, W11, b11, W12, b12, W21, b21, W22, b22):
    src = edge_index[0]
    dst = edge_index[1]
    pad = _EPAD - _E
    src2d = jnp.concatenate(
        [src, jnp.zeros((pad,), jnp.int32)]).reshape(_ROWS, _CHUNK)
    dst2d = jnp.concatenate(
        [dst, jnp.full((pad,), _N, jnp.int32)]).reshape(_ROWS, _CHUNK)
    b11r = b11.reshape(1, _D)
    b12r = b12.reshape(1, _D)
    b21r = b21.reshape(1, _D)
    b22r = b22.reshape(1, _D)
    p1 = _agg(x, src2d, dst2d)
    h = _mlp(p1, x, W11, b11r, W12, b12r, True)
    p2 = _agg(h, src2d, dst2d)
    out = _mlp(p2, h, W21, b21r, W22, b22r, False)
    return out


# speedup vs baseline: 2.8343x; 2.8343x over previous
import functools

import jax
import jax.numpy as jnp
from jax import lax
from jax.experimental import pallas as pl
from jax.experimental.pallas import tpu as pltpu
from jax.experimental.pallas import tpu_sc as plsc

_N = 10000
_D = 128
_E = 320000
_NSUB = 16
_NCORE = 2
_CHUNK = 128
_CPS = 80
_ROWS = _CPS * _NSUB * _NCORE
_EPAD = _ROWS * _CHUNK
_RPS = 624
_RTAIL = _N - _RPS * _NSUB

_sc_mesh = plsc.VectorSubcoreMesh(core_axis_name="c", subcore_axis_name="s")


@functools.partial(
    pl.kernel,
    out_type=jax.ShapeDtypeStruct((_NCORE, _N, _D), jnp.float32),
    mesh=_sc_mesh,
    scratch_types=[
        pltpu.VMEM((_CPS, _CHUNK), jnp.int32),
        pltpu.VMEM((_CPS, _CHUNK), jnp.int32),
        pltpu.VMEM((_CHUNK, _D), jnp.float32),
        pltpu.VMEM_SHARED((_N + 1, _D), jnp.float32),
    ],
)
def _agg(x_hbm, src_hbm, dst_hbm, p_hbm, src_idx, dst_idx, rows, acc):
    cid = lax.axis_index("c")
    sid = lax.axis_index("s")
    cbase = (cid * _NSUB + sid) * _CPS
    pltpu.sync_copy(src_hbm.at[pl.ds(cbase, _CPS)], src_idx)
    pltpu.sync_copy(dst_hbm.at[pl.ds(cbase, _CPS)], dst_idx)
    rbase = sid * _RPS
    pltpu.sync_copy(x_hbm.at[pl.ds(rbase, _RPS)], acc.at[pl.ds(rbase, _RPS)])

    @pl.when(sid == _NSUB - 1)
    def _():
        pltpu.sync_copy(x_hbm.at[pl.ds(_RPS * _NSUB, _RTAIL)],
                        acc.at[pl.ds(_RPS * _NSUB, _RTAIL)])

    plsc.subcore_barrier()

    @pl.loop(0, _CPS)
    def _(j):
        pltpu.sync_copy(x_hbm.at[src_idx.at[j]], rows)
        pltpu.sync_copy(rows, acc.at[dst_idx.at[j]], add=True)

    plsc.subcore_barrier()
    pltpu.sync_copy(acc.at[pl.ds(rbase, _RPS)],
                    p_hbm.at[cid, pl.ds(rbase, _RPS)])

    @pl.when(sid == _NSUB - 1)
    def _():
        pltpu.sync_copy(acc.at[pl.ds(_RPS * _NSUB, _RTAIL)],
                        p_hbm.at[cid, pl.ds(_RPS * _NSUB, _RTAIL)])


def _mlp_block(p_ref, x_ref, w1_ref, b1_ref, w2_ref, b2_ref, o_ref, *,
               final_relu):
    pb = p_ref[...]
    z = pb[0] + pb[1] - x_ref[...]
    t = jnp.dot(z, w1_ref[...], preferred_element_type=jnp.float32)
    t = jnp.maximum(t + b1_ref[...], 0.0)
    o = jnp.dot(t, w2_ref[...], preferred_element_type=jnp.float32)
    o = o + b2_ref[...]
    if final_relu:
        o = jnp.maximum(o, 0.0)
    o_ref[...] = o


_BLK = 1000


def _mlp(p, x, w1, b1, w2, b2, final_relu):
    return pl.pallas_call(
        functools.partial(_mlp_block, final_relu=final_relu),
        grid=(_N // _BLK,),
        in_specs=[
            pl.BlockSpec((_NCORE, _BLK, _D), lambda i: (0, i, 0)),
            pl.BlockSpec((_BLK, _D), lambda i: (i, 0)),
            pl.BlockSpec((_D, _D), lambda i: (0, 0)),
            pl.BlockSpec((1, _D), lambda i: (0, 0)),
            pl.BlockSpec((_D, _D), lambda i: (0, 0)),
            pl.BlockSpec((1, _D), lambda i: (0, 0)),
        ],
        out_specs=pl.BlockSpec((_BLK, _D), lambda i: (i, 0)),
        out_shape=jax.ShapeDtypeStruct((_N, _D), jnp.float32),
    )(p, x, w1, b1, w2, b2)


def kernel(x, edge_index, W11, b11, W12, b12, W21, b21, W22, b22):
    src = edge_index[0]
    dst = edge_index[1]
    pad = _EPAD - _E
    src2d = jnp.concatenate(
        [src, jnp.zeros((pad,), jnp.int32)]).reshape(_ROWS, _CHUNK)
    dst2d = jnp.concatenate(
        [dst, jnp.full((pad,), _N, jnp.int32)]).reshape(_ROWS, _CHUNK)
    b11r = b11.reshape(1, _D)
    b12r = b12.reshape(1, _D)
    b21r = b21.reshape(1, _D)
    b22r = b22.reshape(1, _D)
    p1 = _agg(x, src2d, dst2d)
    h = _mlp(p1, x, W11, b11r, W12, b12r, True)
    p2 = _agg(h, src2d, dst2d)
    out = _mlp(p2, h, W21, b21r, W22, b22r, False)
    return out


# speedup vs baseline: 3.1621x; 1.1156x over previous
import functools

import jax
import jax.numpy as jnp
from jax import lax
from jax.experimental import pallas as pl
from jax.experimental.pallas import tpu as pltpu
from jax.experimental.pallas import tpu_sc as plsc

_N = 10000
_D = 128
_E = 320000
_NSUB = 16
_NCORE = 2
_CHUNK = 128
_CPS = 80
_ROWS = _CPS * _NSUB * _NCORE
_EPAD = _ROWS * _CHUNK
_RPS = 624
_RTAIL = _N - _RPS * _NSUB

_sc_mesh = plsc.VectorSubcoreMesh(core_axis_name="c", subcore_axis_name="s")


@functools.partial(
    pl.kernel,
    out_type=jax.ShapeDtypeStruct((_NCORE, _N, _D), jnp.float32),
    mesh=_sc_mesh,
    scratch_types=[
        pltpu.VMEM((_CPS // 2, _CHUNK), jnp.int32),
        pltpu.VMEM((_CPS // 2, _CHUNK), jnp.int32),
        pltpu.VMEM((_CHUNK, _D), jnp.float32),
        pltpu.VMEM((_CHUNK, _D), jnp.float32),
        pltpu.VMEM_SHARED((_N + 1, _D), jnp.float32),
        pltpu.SemaphoreType.DMA,
        pltpu.SemaphoreType.DMA,
    ],
)
def _agg(x_hbm, src_hbm, dst_hbm, p_hbm, src_idx, dst_idx, rows0, rows1,
         acc, sem0, sem1):
    cid = lax.axis_index("c")
    sid = lax.axis_index("s")
    cbase = (cid * _NSUB + sid) * _CPS
    rbase = sid * _RPS
    pltpu.sync_copy(x_hbm.at[pl.ds(rbase, _RPS)], acc.at[pl.ds(rbase, _RPS)])

    @pl.when(sid == _NSUB - 1)
    def _():
        pltpu.sync_copy(x_hbm.at[pl.ds(_RPS * _NSUB, _RTAIL)],
                        acc.at[pl.ds(_RPS * _NSUB, _RTAIL)])

    plsc.subcore_barrier()

    half_rows = _CPS // 2

    @pl.loop(0, 2)
    def _(half):
        base = cbase + half * half_rows
        pltpu.sync_copy(src_hbm.at[pl.ds(base, half_rows)], src_idx)
        pltpu.sync_copy(dst_hbm.at[pl.ds(base, half_rows)], dst_idx)
        pltpu.async_copy(x_hbm.at[src_idx.at[0]], rows0, sem0)
        pltpu.async_copy(x_hbm.at[src_idx.at[1]], rows1, sem1)

        @pl.loop(0, half_rows - 2, step=2)
        def _(j):
            pltpu.make_async_copy(x_hbm.at[src_idx.at[j]], rows0, sem0).wait()
            pltpu.sync_copy(rows0, acc.at[dst_idx.at[j]], add=True)
            pltpu.async_copy(x_hbm.at[src_idx.at[j + 2]], rows0, sem0)
            pltpu.make_async_copy(
                x_hbm.at[src_idx.at[j + 1]], rows1, sem1).wait()
            pltpu.sync_copy(rows1, acc.at[dst_idx.at[j + 1]], add=True)
            pltpu.async_copy(x_hbm.at[src_idx.at[j + 3]], rows1, sem1)

        pltpu.make_async_copy(
            x_hbm.at[src_idx.at[half_rows - 2]], rows0, sem0).wait()
        pltpu.sync_copy(rows0, acc.at[dst_idx.at[half_rows - 2]], add=True)
        pltpu.make_async_copy(
            x_hbm.at[src_idx.at[half_rows - 1]], rows1, sem1).wait()
        pltpu.sync_copy(rows1, acc.at[dst_idx.at[half_rows - 1]], add=True)

    plsc.subcore_barrier()
    pltpu.sync_copy(acc.at[pl.ds(rbase, _RPS)],
                    p_hbm.at[cid, pl.ds(rbase, _RPS)])

    @pl.when(sid == _NSUB - 1)
    def _():
        pltpu.sync_copy(acc.at[pl.ds(_RPS * _NSUB, _RTAIL)],
                        p_hbm.at[cid, pl.ds(_RPS * _NSUB, _RTAIL)])


def _mlp_block(p_ref, x_ref, w1_ref, b1_ref, w2_ref, b2_ref, o_ref, *,
               final_relu):
    pb = p_ref[...]
    z = pb[0] + pb[1] - x_ref[...]
    t = jnp.dot(z, w1_ref[...], preferred_element_type=jnp.float32)
    t = jnp.maximum(t + b1_ref[...], 0.0)
    o = jnp.dot(t, w2_ref[...], preferred_element_type=jnp.float32)
    o = o + b2_ref[...]
    if final_relu:
        o = jnp.maximum(o, 0.0)
    o_ref[...] = o


_BLK = 1000


def _mlp(p, x, w1, b1, w2, b2, final_relu):
    return pl.pallas_call(
        functools.partial(_mlp_block, final_relu=final_relu),
        grid=(_N // _BLK,),
        in_specs=[
            pl.BlockSpec((_NCORE, _BLK, _D), lambda i: (0, i, 0)),
            pl.BlockSpec((_BLK, _D), lambda i: (i, 0)),
            pl.BlockSpec((_D, _D), lambda i: (0, 0)),
            pl.BlockSpec((1, _D), lambda i: (0, 0)),
            pl.BlockSpec((_D, _D), lambda i: (0, 0)),
            pl.BlockSpec((1, _D), lambda i: (0, 0)),
        ],
        out_specs=pl.BlockSpec((_BLK, _D), lambda i: (i, 0)),
        out_shape=jax.ShapeDtypeStruct((_N, _D), jnp.float32),
    )(p, x, w1, b1, w2, b2)


def kernel(x, edge_index, W11, b11, W12, b12, W21, b21, W22, b22):
    src = edge_index[0]
    dst = edge_index[1]
    pad = _EPAD - _E
    src2d = jnp.concatenate(
        [src, jnp.zeros((pad,), jnp.int32)]).reshape(_ROWS, _CHUNK)
    dst2d = jnp.concatenate(
        [dst, jnp.full((pad,), _N, jnp.int32)]).reshape(_ROWS, _CHUNK)
    b11r = b11.reshape(1, _D)
    b12r = b12.reshape(1, _D)
    b21r = b21.reshape(1, _D)
    b22r = b22.reshape(1, _D)
    p1 = _agg(x, src2d, dst2d)
    h = _mlp(p1, x, W11, b11r, W12, b12r, True)
    p2 = _agg(h, src2d, dst2d)
    out = _mlp(p2, h, W21, b21r, W22, b22r, False)
    return out
